# baseline (speedup 1.0000x reference)
import jax
import jax.numpy as jnp
from jax.experimental import pallas as pl
from jax.experimental.pallas import tpu as pltpu
from jax.experimental.pallas import tpu_sc as plsc

DIM = 64
WINDOW = 512


def _table_to_rowmajor(table_t):
    nv = table_t.shape[1]
    vblk = 1024

    def body(x_ref, o_ref):
        y = x_ref[...].T
        o_ref[...] = jnp.concatenate([y, y], axis=1)

    return pl.pallas_call(
        body,
        grid=(-(-nv // vblk),),
        in_specs=[pl.BlockSpec((DIM, vblk), lambda j: (0, j))],
        out_specs=pl.BlockSpec((vblk, 2 * DIM), lambda j: (j, 0)),
        out_shape=jax.ShapeDtypeStruct((nv, 2 * DIM), table_t.dtype),
        compiler_params=pltpu.CompilerParams(
            dimension_semantics=("parallel",)
        ),
    )(table_t)


def _transpose_out(flat):
    nb, nf = flat.shape
    bblk, fblk = 1024, 128

    def body(x_ref, o_ref):
        o_ref[...] = x_ref[...].T

    return pl.pallas_call(
        body,
        grid=(nb // bblk, nf // fblk),
        in_specs=[pl.BlockSpec((bblk, fblk), lambda i, j: (i, j))],
        out_specs=pl.BlockSpec((fblk, bblk), lambda i, j: (j, i)),
        out_shape=jax.ShapeDtypeStruct((nf, nb), flat.dtype),
        compiler_params=pltpu.CompilerParams(
            dimension_semantics=("parallel", "parallel")
        ),
    )(flat)


def _sc_gather(table, idx_flat):
    n = idx_flat.shape[1]
    mesh = plsc.VectorSubcoreMesh(core_axis_name="c", subcore_axis_name="s")

    @pl.kernel(
        out_type=jax.ShapeDtypeStruct((n, DIM), table.dtype),
        mesh=mesh,
        compiler_params=pltpu.CompilerParams(use_tc_tiling_on_sc=False),
    )
    def k(table_hbm, idx_hbm, out_hbm):
        def body(i_vmem, o_vmem):
            pltpu.sync_copy(table_hbm.at[i_vmem.at[0]], o_vmem)

        pltpu.emit_pipeline(
            body,
            grid=(n // WINDOW,),
            in_specs=[pl.BlockSpec((1, WINDOW), index_map=lambda i: (0, i))],
            out_specs=[pl.BlockSpec((WINDOW, DIM), index_map=lambda i: (i, 0))],
            core_axis_name=("c", "s"),
            dimension_semantics=(pltpu.PARALLEL,),
        )(idx_hbm, out_hbm)

    return k(table, idx_flat)


def kernel(idxs, emb_weight):
    b, s = idxs.shape
    n = b * s
    nv = emb_weight.shape[0]
    idx_flat = (idxs * 2).reshape(1, n)
    table_lin = _table_to_rowmajor(emb_weight.T).reshape(2 * nv, DIM)
    out = _sc_gather(table_lin, idx_flat)
    out_t = _transpose_out(out.reshape(b, s * DIM))
    return out_t.reshape(s, DIM, b).transpose(2, 0, 1)


# speedup vs baseline: 1.6120x; 1.6120x over previous
import jax
import jax.numpy as jnp
from jax.experimental import pallas as pl
from jax.experimental.pallas import tpu as pltpu
from jax.experimental.pallas import tpu_sc as plsc

DIM = 64
WINDOW = 512


def _table_to_rowmajor(table_t):
    nv = table_t.shape[1]
    vblk = 4096

    def body(x_ref, o_ref):
        y = x_ref[...].T
        o_ref[...] = jnp.concatenate([y, y], axis=1)

    return pl.pallas_call(
        body,
        grid=(-(-nv // vblk),),
        in_specs=[pl.BlockSpec((DIM, vblk), lambda j: (0, j))],
        out_specs=pl.BlockSpec((vblk, 2 * DIM), lambda j: (j, 0)),
        out_shape=jax.ShapeDtypeStruct((nv, 2 * DIM), table_t.dtype),
        compiler_params=pltpu.CompilerParams(
            dimension_semantics=("parallel",)
        ),
    )(table_t)


def _transpose_out(flat):
    nb, nf = flat.shape
    bblk, fblk = 2048, 640

    def body(x_ref, o_ref):
        o_ref[...] = x_ref[...].T

    return pl.pallas_call(
        body,
        grid=(nb // bblk, nf // fblk),
        in_specs=[pl.BlockSpec((bblk, fblk), lambda i, j: (i, j))],
        out_specs=pl.BlockSpec((fblk, bblk), lambda i, j: (j, i)),
        out_shape=jax.ShapeDtypeStruct((nf, nb), flat.dtype),
        compiler_params=pltpu.CompilerParams(
            dimension_semantics=("parallel", "parallel")
        ),
    )(flat)


def _sc_gather(table, idx_flat):
    n = idx_flat.shape[1]
    mesh = plsc.VectorSubcoreMesh(core_axis_name="c", subcore_axis_name="s")

    @pl.kernel(
        out_type=jax.ShapeDtypeStruct((n, DIM), table.dtype),
        mesh=mesh,
        compiler_params=pltpu.CompilerParams(use_tc_tiling_on_sc=False),
    )
    def k(table_hbm, idx_hbm, out_hbm):
        def body(i_vmem, o_vmem):
            pltpu.sync_copy(table_hbm.at[i_vmem.at[0]], o_vmem)

        pltpu.emit_pipeline(
            body,
            grid=(n // WINDOW,),
            in_specs=[pl.BlockSpec((1, WINDOW), index_map=lambda i: (0, i))],
            out_specs=[pl.BlockSpec((WINDOW, DIM), index_map=lambda i: (i, 0))],
            core_axis_name=("c", "s"),
            dimension_semantics=(pltpu.PARALLEL,),
        )(idx_hbm, out_hbm)

    return k(table, idx_flat)


def kernel(idxs, emb_weight):
    b, s = idxs.shape
    n = b * s
    nv = emb_weight.shape[0]
    idx_flat = (idxs * 2).reshape(1, n)
    table_lin = _table_to_rowmajor(emb_weight.T).reshape(2 * nv, DIM)
    out = _sc_gather(table_lin, idx_flat)
    out_t = _transpose_out(out.reshape(b, s * DIM))
    return out_t.reshape(s, DIM, b).transpose(2, 0, 1)


# speedup vs baseline: 1.7455x; 1.0828x over previous
import jax
import jax.numpy as jnp
from jax.experimental import pallas as pl
from jax.experimental.pallas import tpu as pltpu
from jax.experimental.pallas import tpu_sc as plsc

DIM = 64
WINDOW = 800


def _table_to_rowmajor(table_t):
    nv = table_t.shape[1]
    vblk = 8192

    def body(x_ref, o_ref):
        y = x_ref[...].T
        o_ref[...] = jnp.concatenate([y, y], axis=1)

    return pl.pallas_call(
        body,
        grid=(-(-nv // vblk),),
        in_specs=[pl.BlockSpec((DIM, vblk), lambda j: (0, j))],
        out_specs=pl.BlockSpec((vblk, 2 * DIM), lambda j: (j, 0)),
        out_shape=jax.ShapeDtypeStruct((nv, 2 * DIM), table_t.dtype),
        compiler_params=pltpu.CompilerParams(
            dimension_semantics=("parallel",)
        ),
    )(table_t)


def _transpose_out(flat):
    nb, nf = flat.shape
    bblk, fblk = 2048, 640

    def body(x_ref, o_ref):
        o_ref[...] = x_ref[...].T

    return pl.pallas_call(
        body,
        grid=(nb // bblk, nf // fblk),
        in_specs=[pl.BlockSpec((bblk, fblk), lambda i, j: (i, j))],
        out_specs=pl.BlockSpec((fblk, bblk), lambda i, j: (j, i)),
        out_shape=jax.ShapeDtypeStruct((nf, nb), flat.dtype),
        compiler_params=pltpu.CompilerParams(
            dimension_semantics=("parallel", "parallel")
        ),
    )(flat)


def _sc_gather(table, idx_flat):
    n = idx_flat.shape[1]
    mesh = plsc.VectorSubcoreMesh(core_axis_name="c", subcore_axis_name="s")

    @pl.kernel(
        out_type=jax.ShapeDtypeStruct((n, DIM), table.dtype),
        mesh=mesh,
        compiler_params=pltpu.CompilerParams(use_tc_tiling_on_sc=False),
    )
    def k(table_hbm, idx_hbm, out_hbm):
        def body(i_vmem, o_vmem):
            pltpu.sync_copy(table_hbm.at[i_vmem.at[0]], o_vmem)

        pltpu.emit_pipeline(
            body,
            grid=(n // WINDOW,),
            in_specs=[pl.BlockSpec((1, WINDOW), index_map=lambda i: (0, i))],
            out_specs=[pl.BlockSpec((WINDOW, DIM), index_map=lambda i: (i, 0))],
            core_axis_name=("c", "s"),
            dimension_semantics=(pltpu.PARALLEL,),
        )(idx_hbm, out_hbm)

    return k(table, idx_flat)


def kernel(idxs, emb_weight):
    b, s = idxs.shape
    n = b * s
    nv = emb_weight.shape[0]
    idx_flat = (idxs * 2).reshape(1, n)
    table_lin = _table_to_rowmajor(emb_weight.T).reshape(2 * nv, DIM)
    out = _sc_gather(table_lin, idx_flat)
    out_t = _transpose_out(out.reshape(b, s * DIM))
    return out_t.reshape(s, DIM, b).transpose(2, 0, 1)


# speedup vs baseline: 1.8056x; 1.0345x over previous
import jax
import jax.numpy as jnp
from jax.experimental import pallas as pl
from jax.experimental.pallas import tpu as pltpu
from jax.experimental.pallas import tpu_sc as plsc

DIM = 64
WINDOW = 800


def _table_to_rowmajor(table_t):
    nv = table_t.shape[1]
    vblk = 8192

    def body(x_ref, o_ref):
        y = x_ref[...].T
        o_ref[...] = jnp.concatenate([y, y], axis=1)

    return pl.pallas_call(
        body,
        grid=(-(-nv // vblk),),
        in_specs=[pl.BlockSpec((DIM, vblk), lambda j: (0, j))],
        out_specs=pl.BlockSpec((vblk, 2 * DIM), lambda j: (j, 0)),
        out_shape=jax.ShapeDtypeStruct((nv, 2 * DIM), table_t.dtype),
        compiler_params=pltpu.CompilerParams(
            dimension_semantics=("parallel",)
        ),
    )(table_t)


def _transpose_chunk(flat_c, prev, c, nb_total):
    nbc, nf = flat_c.shape
    bblk, fblk = 2048, 640
    col_off = c * (nbc // bblk)

    def body(x_ref, *rest):
        o_ref = rest[-1]
        o_ref[...] = x_ref[...].T

    in_specs = [pl.BlockSpec((bblk, fblk), lambda i, j: (i, j))]
    args = (flat_c,)
    aliases = {}
    if prev is not None:
        in_specs.append(pl.BlockSpec(memory_space=pl.ANY))
        args = (flat_c, prev)
        aliases = {1: 0}

    return pl.pallas_call(
        body,
        grid=(nbc // bblk, nf // fblk),
        in_specs=in_specs,
        out_specs=pl.BlockSpec((fblk, bblk), lambda i, j: (j, i + col_off)),
        out_shape=jax.ShapeDtypeStruct((nf, nb_total), flat_c.dtype),
        input_output_aliases=aliases,
    )(*args)


def _sc_gather(table, idx_flat):
    n = idx_flat.shape[1]
    mesh = plsc.VectorSubcoreMesh(core_axis_name="c", subcore_axis_name="s")

    @pl.kernel(
        out_type=jax.ShapeDtypeStruct((n, DIM), table.dtype),
        mesh=mesh,
        compiler_params=pltpu.CompilerParams(use_tc_tiling_on_sc=False),
    )
    def k(table_hbm, idx_hbm, out_hbm):
        def body(i_vmem, o_vmem):
            pltpu.sync_copy(table_hbm.at[i_vmem.at[0]], o_vmem)

        pltpu.emit_pipeline(
            body,
            grid=(n // WINDOW,),
            in_specs=[pl.BlockSpec((1, WINDOW), index_map=lambda i: (0, i))],
            out_specs=[pl.BlockSpec((WINDOW, DIM), index_map=lambda i: (i, 0))],
            core_axis_name=("c", "s"),
            dimension_semantics=(pltpu.PARALLEL,),
        )(idx_hbm, out_hbm)

    return k(table, idx_flat)


def kernel(idxs, emb_weight):
    b, s = idxs.shape
    nv = emb_weight.shape[0]
    nchunks = 4
    bc = b // nchunks
    table_lin = _table_to_rowmajor(emb_weight.T).reshape(2 * nv, DIM)
    idx2 = idxs * 2
    out_t = None
    for c in range(nchunks):
        idx_c = idx2[c * bc : (c + 1) * bc].reshape(1, bc * s)
        g_c = _sc_gather(table_lin, idx_c)
        out_t = _transpose_chunk(g_c.reshape(bc, s * DIM), out_t, c, b)
    return out_t.reshape(s, DIM, b).transpose(2, 0, 1)


# speedup vs baseline: 1.8829x; 1.0428x over previous
import jax
import jax.numpy as jnp
from jax.experimental import pallas as pl
from jax.experimental.pallas import tpu as pltpu
from jax.experimental.pallas import tpu_sc as plsc

DIM = 64
WINDOW = 800


def _table_to_rowmajor(table_t):
    nv = table_t.shape[1]
    vblk = 16384

    def body(x_ref, o_ref):
        y = x_ref[...].T
        o_ref[...] = jnp.concatenate([y, y], axis=1)

    return pl.pallas_call(
        body,
        grid=(-(-nv // vblk),),
        in_specs=[pl.BlockSpec((DIM, vblk), lambda j: (0, j))],
        out_specs=pl.BlockSpec((vblk, 2 * DIM), lambda j: (j, 0)),
        out_shape=jax.ShapeDtypeStruct((nv, 2 * DIM), table_t.dtype),
        compiler_params=pltpu.CompilerParams(
            dimension_semantics=("parallel",)
        ),
    )(table_t)


def _transpose_chunk(flat_c, prev, c, nb_total):
    nbc, nf = flat_c.shape
    bblk, fblk = 2048, 640
    col_off = c * (nbc // bblk)

    def body(x_ref, *rest):
        o_ref = rest[-1]
        o_ref[...] = x_ref[...].T

    in_specs = [pl.BlockSpec((bblk, fblk), lambda i, j: (i, j))]
    args = (flat_c,)
    aliases = {}
    if prev is not None:
        in_specs.append(pl.BlockSpec(memory_space=pl.ANY))
        args = (flat_c, prev)
        aliases = {1: 0}

    return pl.pallas_call(
        body,
        grid=(nbc // bblk, nf // fblk),
        in_specs=in_specs,
        out_specs=pl.BlockSpec((fblk, bblk), lambda i, j: (j, i + col_off)),
        out_shape=jax.ShapeDtypeStruct((nf, nb_total), flat_c.dtype),
        input_output_aliases=aliases,
    )(*args)


def _sc_gather(table, idx_flat):
    n = idx_flat.shape[1]
    mesh = plsc.VectorSubcoreMesh(core_axis_name="c", subcore_axis_name="s")

    @pl.kernel(
        out_type=jax.ShapeDtypeStruct((n, DIM), table.dtype),
        mesh=mesh,
        compiler_params=pltpu.CompilerParams(use_tc_tiling_on_sc=False),
    )
    def k(table_hbm, idx_hbm, out_hbm):
        def body(i_vmem, o_vmem):
            pltpu.sync_copy(table_hbm.at[i_vmem.at[0]], o_vmem)

        pltpu.emit_pipeline(
            body,
            grid=(n // WINDOW,),
            in_specs=[pl.BlockSpec((1, WINDOW), index_map=lambda i: (0, i))],
            out_specs=[pl.BlockSpec((WINDOW, DIM), index_map=lambda i: (i, 0))],
            core_axis_name=("c", "s"),
            dimension_semantics=(pltpu.PARALLEL,),
        )(idx_hbm, out_hbm)

    return k(table, idx_flat)


def kernel(idxs, emb_weight):
    b, s = idxs.shape
    nv = emb_weight.shape[0]
    nchunks = 4
    bc = b // nchunks
    table_lin = _table_to_rowmajor(emb_weight.T).reshape(2 * nv, DIM)
    idx2 = idxs * 2
    out_t = None
    for c in range(nchunks):
        idx_c = idx2[c * bc : (c + 1) * bc].reshape(1, bc * s)
        g_c = _sc_gather(table_lin, idx_c)
        out_t = _transpose_chunk(g_c.reshape(bc, s * DIM), out_t, c, b)
    return out_t.reshape(s, DIM, b).transpose(2, 0, 1)
